# baseline (speedup 1.0000x reference)
import functools
import math

import jax
import jax.numpy as jnp
from jax import lax
from jax.experimental import pallas as pl
from jax.experimental.pallas import tpu as pltpu
from jax.experimental.pallas import tpu_sc as plsc

NUM_JOINTS = 21
IMG_SIZE = 256
G_SIZE = 11
G_SIGMA = 3.0
BATCH = 32

_C = BATCH * NUM_JOINTS
_INV2S2 = 1.0 / (2.0 * G_SIGMA * G_SIGMA)
_R = (G_SIZE - 1) // 2

_NC, _NS, _L = 2, 16, 16
_NW = _NC * _NS
_CPW = _C // _NW
_NP = G_SIZE * G_SIZE
_NQ = 8


def _sc_body(cpw, xy_hbm, out_hbm, xy_v, img):
    S = IMG_SIZE
    wid = lax.axis_index("s") * _NC + lax.axis_index("c")
    pltpu.sync_copy(xy_hbm.at[wid], xy_v)

    lane_i = lax.iota(jnp.int32, _L)
    zero16 = lane_i.astype(jnp.float32) * 0.0

    dys, dxs, gvs, oks = [], [], [], []
    for q in range(_NQ):
        p = lane_i + (q * _L)
        dy = p // G_SIZE - _R
        dx = p % G_SIZE - _R
        ok = p < _NP
        dyf = dy.astype(jnp.float32)
        dxf = dx.astype(jnp.float32)
        gv = jnp.where(ok, jnp.exp((dyf * dyf + dxf * dxf) * (-_INV2S2)), 0.0)
        dys.append(dy)
        dxs.append(dx)
        gvs.append(gv)
        oks.append(ok)

    def zrow(r, carry):
        for chk in range(S // _L):
            img[r, pl.ds(chk * _L, _L)] = zero16
        return carry

    lax.fori_loop(0, S, zrow, 0)

    def rnd_clamp(v):
        u = jnp.clip((v * 0.25 + 0.5) * (S - 1), 0.0, float(S - 1))
        c0 = (u + 0.5).astype(jnp.int32)
        tie = (c0.astype(jnp.float32) - u) == 0.5
        odd = (c0 & 1) == 1
        return c0 - jnp.where(tie & odd, 1, 0)

    def chan(i, carry):
        ix = rnd_clamp(xy_v[i, 0])
        iy = rnd_clamp(xy_v[i, 1])

        rows, cols, masks = [], [], []
        for q in range(_NQ):
            row = iy + dys[q]
            col = ix + dxs[q]
            m = (oks[q]
                 & (row >= 0) & (row <= S - 1)
                 & (col >= 0) & (col <= S - 1))
            plsc.store_scatter(img, [row, col], gvs[q], mask=m)
            rows.append(row)
            cols.append(col)
            masks.append(m)

        pltpu.sync_copy(img, out_hbm.at[i * _NW + wid])

        for q in range(_NQ):
            plsc.store_scatter(img, [rows[q], cols[q]], zero16, mask=masks[q])

        return carry

    lax.fori_loop(0, cpw, chan, 0)


def _sc_heatmap(xy_splat, nch):
    cpw = nch // _NW
    mesh = plsc.VectorSubcoreMesh(core_axis_name="c", subcore_axis_name="s")
    return pl.kernel(
        functools.partial(_sc_body, cpw),
        out_type=jax.ShapeDtypeStruct((nch, IMG_SIZE, IMG_SIZE), jnp.float32),
        mesh=mesh,
        scratch_types=[
            pltpu.VMEM((cpw, 2, _L), jnp.float32),
            pltpu.VMEM((IMG_SIZE, IMG_SIZE), jnp.float32),
        ],
        compiler_params=pltpu.CompilerParams(needs_layout_passes=False),
    )(xy_splat)


def kernel(joint):
    B, J, _ = joint.shape
    xy = joint.reshape(B * J, 3)[:, :2]
    xy_splat = jnp.broadcast_to(xy[:, :, None], (_C, 2, _L))
    xy_splat = xy_splat.reshape(_CPW, _NW, 2, _L).swapaxes(0, 1)
    out = _sc_heatmap(xy_splat, _C)
    return out.reshape(B, J, IMG_SIZE, IMG_SIZE)
